# baseline (speedup 1.0000x reference)
import functools

import jax
import jax.numpy as jnp
from jax.experimental import pallas as pl
from jax.experimental.pallas import tpu as pltpu
from jax.experimental.pallas import tpu_sc as plsc

EMB_DIM = 64
HID_DIM = 128
GATHER_WINDOW = 128
MLP_BLOCK = 2048


def _sc_double_gather(user_table, item_table, uids, iids):
    batch = uids.shape[-1]
    grid = batch // GATHER_WINDOW
    mesh = plsc.VectorSubcoreMesh(core_axis_name="c", subcore_axis_name="s")

    @functools.partial(
        pl.kernel,
        out_type=(
            jax.ShapeDtypeStruct((batch, EMB_DIM), jnp.float32),
            jax.ShapeDtypeStruct((batch, EMB_DIM), jnp.float32),
        ),
        mesh=mesh,
        compiler_params=pltpu.CompilerParams(use_tc_tiling_on_sc=False),
    )
    def gather_kernel(ut_hbm, it_hbm, ui_hbm, ii_hbm, uo_hbm, io_hbm):
        def body(ui_v, ii_v, uo_v, io_v):
            pltpu.sync_copy(ut_hbm.at[ui_v.at[0]], uo_v)
            pltpu.sync_copy(it_hbm.at[ii_v.at[0]], io_v)

        pltpu.emit_pipeline(
            body,
            grid=(grid,),
            in_specs=[
                pl.BlockSpec((1, GATHER_WINDOW), lambda i: (0, i)),
                pl.BlockSpec((1, GATHER_WINDOW), lambda i: (0, i)),
            ],
            out_specs=[
                pl.BlockSpec((GATHER_WINDOW, EMB_DIM), lambda i: (i, 0)),
                pl.BlockSpec((GATHER_WINDOW, EMB_DIM), lambda i: (i, 0)),
            ],
            core_axis_name=("c", "s"),
            dimension_semantics=(pltpu.PARALLEL,),
        )(ui_hbm, ii_hbm, uo_hbm, io_hbm)

    return gather_kernel(user_table, item_table, uids, iids)


def _mlp_body(ue_ref, ie_ref, w1u_ref, w1i_ref, b1_ref, w2_ref, b2_ref,
              w3_ref, b3_ref, out_ref):
    h = jnp.dot(ue_ref[...], w1u_ref[...], preferred_element_type=jnp.float32)
    h += jnp.dot(ie_ref[...], w1i_ref[...], preferred_element_type=jnp.float32)
    h = jnp.maximum(h + b1_ref[...], 0.0)
    h = jnp.dot(h, w2_ref[...], preferred_element_type=jnp.float32)
    h = jnp.maximum(h + b2_ref[...], 0.0)
    logit = jnp.sum(h * w3_ref[...], axis=-1) + b3_ref[0]
    p = jax.nn.sigmoid(logit)
    out_ref[...] = jnp.clip(p, 0.01, 0.99)


def _tc_mlp(user_emb, item_emb, W1, b1, W2, b2, W3, b3):
    batch = user_emb.shape[0]
    w1u = W1[:EMB_DIM]
    w1i = W1[EMB_DIM:]
    w3r = jnp.reshape(W3, (1, HID_DIM // 2))
    b1r = jnp.reshape(b1, (1, HID_DIM))
    b2r = jnp.reshape(b2, (1, HID_DIM // 2))
    grid = batch // MLP_BLOCK
    rep = lambda i: (0, 0)
    return pl.pallas_call(
        _mlp_body,
        grid=(grid,),
        in_specs=[
            pl.BlockSpec((MLP_BLOCK, EMB_DIM), lambda i: (i, 0)),
            pl.BlockSpec((MLP_BLOCK, EMB_DIM), lambda i: (i, 0)),
            pl.BlockSpec((EMB_DIM, HID_DIM), rep),
            pl.BlockSpec((EMB_DIM, HID_DIM), rep),
            pl.BlockSpec((1, HID_DIM), rep),
            pl.BlockSpec((HID_DIM, HID_DIM // 2), rep),
            pl.BlockSpec((1, HID_DIM // 2), rep),
            pl.BlockSpec((1, HID_DIM // 2), rep),
            pl.BlockSpec((1,), lambda i: (0,)),
        ],
        out_specs=pl.BlockSpec((MLP_BLOCK,), lambda i: (i,)),
        out_shape=jax.ShapeDtypeStruct((batch,), jnp.float32),
    )(user_emb, item_emb, w1u, w1i, b1r, W2, b2r, w3r, b3)


def kernel(user_ids, item_ids, user_table, item_table, W1, b1, W2, b2, W3, b3):
    batch = user_ids.shape[0]
    uids = jnp.reshape(user_ids.astype(jnp.int32), (1, batch))
    iids = jnp.reshape(item_ids.astype(jnp.int32), (1, batch))
    user_emb, item_emb = _sc_double_gather(user_table, item_table, uids, iids)
    return _tc_mlp(user_emb, item_emb, W1, b1, W2, b2, W3, b3)


# speedup vs baseline: 1.0155x; 1.0155x over previous
import functools

import jax
import jax.numpy as jnp
from jax.experimental import pallas as pl
from jax.experimental.pallas import tpu as pltpu
from jax.experimental.pallas import tpu_sc as plsc

EMB_DIM = 64
HID_DIM = 128
GATHER_WINDOW = 128
MLP_BLOCK = 2048


def _sc_double_gather(user_pairs, item_pairs, uids, iids):
    batch = uids.shape[-1]
    grid = batch // GATHER_WINDOW
    mesh = plsc.VectorSubcoreMesh(core_axis_name="c", subcore_axis_name="s")

    @functools.partial(
        pl.kernel,
        out_type=(
            jax.ShapeDtypeStruct((batch, 2 * EMB_DIM), jnp.float32),
            jax.ShapeDtypeStruct((batch, 2 * EMB_DIM), jnp.float32),
        ),
        mesh=mesh,
    )
    def gather_kernel(ut_hbm, it_hbm, ui_hbm, ii_hbm, uo_hbm, io_hbm):
        def body(ui_v, ii_v, uo_v, io_v):
            pltpu.sync_copy(ut_hbm.at[ui_v.at[0]], uo_v)
            pltpu.sync_copy(it_hbm.at[ii_v.at[0]], io_v)

        pltpu.emit_pipeline(
            body,
            grid=(grid,),
            in_specs=[
                pl.BlockSpec((1, GATHER_WINDOW), lambda i: (0, i)),
                pl.BlockSpec((1, GATHER_WINDOW), lambda i: (0, i)),
            ],
            out_specs=[
                pl.BlockSpec((GATHER_WINDOW, 2 * EMB_DIM), lambda i: (i, 0)),
                pl.BlockSpec((GATHER_WINDOW, 2 * EMB_DIM), lambda i: (i, 0)),
            ],
            core_axis_name=("c", "s"),
            dimension_semantics=(pltpu.PARALLEL,),
        )(ui_hbm, ii_hbm, uo_hbm, io_hbm)

    return gather_kernel(user_pairs, item_pairs, uids, iids)


def _mlp_body(up_ref, ip_ref, upar_ref, ipar_ref, w1u_ref, w1i_ref, b1_ref,
              w2_ref, b2_ref, w3_ref, b3_ref, out_ref):
    up = up_ref[...]
    ip = ip_ref[...]
    ue = jnp.where(upar_ref[...][:, None] > 0, up[:, EMB_DIM:], up[:, :EMB_DIM])
    ie = jnp.where(ipar_ref[...][:, None] > 0, ip[:, EMB_DIM:], ip[:, :EMB_DIM])
    h = jnp.dot(ue, w1u_ref[...], preferred_element_type=jnp.float32)
    h += jnp.dot(ie, w1i_ref[...], preferred_element_type=jnp.float32)
    h = jnp.maximum(h + b1_ref[...], 0.0)
    h = jnp.dot(h, w2_ref[...], preferred_element_type=jnp.float32)
    h = jnp.maximum(h + b2_ref[...], 0.0)
    logit = jnp.sum(h * w3_ref[...], axis=-1) + b3_ref[0]
    p = jax.nn.sigmoid(logit)
    out_ref[...] = jnp.clip(p, 0.01, 0.99)


def _tc_mlp(user_pairs, item_pairs, upar, ipar, W1, b1, W2, b2, W3, b3):
    batch = user_pairs.shape[0]
    w1u = W1[:EMB_DIM]
    w1i = W1[EMB_DIM:]
    w3r = jnp.reshape(W3, (1, HID_DIM // 2))
    b1r = jnp.reshape(b1, (1, HID_DIM))
    b2r = jnp.reshape(b2, (1, HID_DIM // 2))
    grid = batch // MLP_BLOCK
    rep = lambda i: (0, 0)
    return pl.pallas_call(
        _mlp_body,
        grid=(grid,),
        in_specs=[
            pl.BlockSpec((MLP_BLOCK, 2 * EMB_DIM), lambda i: (i, 0)),
            pl.BlockSpec((MLP_BLOCK, 2 * EMB_DIM), lambda i: (i, 0)),
            pl.BlockSpec((MLP_BLOCK,), lambda i: (i,)),
            pl.BlockSpec((MLP_BLOCK,), lambda i: (i,)),
            pl.BlockSpec((EMB_DIM, HID_DIM), rep),
            pl.BlockSpec((EMB_DIM, HID_DIM), rep),
            pl.BlockSpec((1, HID_DIM), rep),
            pl.BlockSpec((HID_DIM, HID_DIM // 2), rep),
            pl.BlockSpec((1, HID_DIM // 2), rep),
            pl.BlockSpec((1, HID_DIM // 2), rep),
            pl.BlockSpec((1,), lambda i: (0,)),
        ],
        out_specs=pl.BlockSpec((MLP_BLOCK,), lambda i: (i,)),
        out_shape=jax.ShapeDtypeStruct((batch,), jnp.float32),
    )(user_pairs, item_pairs, upar, ipar, w1u, w1i, b1r, W2, b2r, w3r, b3)


def kernel(user_ids, item_ids, user_table, item_table, W1, b1, W2, b2, W3, b3):
    batch = user_ids.shape[0]
    uids32 = user_ids.astype(jnp.int32)
    iids32 = item_ids.astype(jnp.int32)
    upairs_idx = jnp.reshape(uids32 >> 1, (1, batch))
    ipairs_idx = jnp.reshape(iids32 >> 1, (1, batch))
    upar = (uids32 & 1).astype(jnp.float32)
    ipar = (iids32 & 1).astype(jnp.float32)
    ut2 = jnp.reshape(user_table, (user_table.shape[0] // 2, 2 * EMB_DIM))
    it2 = jnp.reshape(item_table, (item_table.shape[0] // 2, 2 * EMB_DIM))
    user_pairs, item_pairs = _sc_double_gather(ut2, it2, upairs_idx, ipairs_idx)
    return _tc_mlp(user_pairs, item_pairs, upar, ipar, W1, b1, W2, b2, W3, b3)
